# baseline (speedup 1.0000x reference)
import functools

import jax
import jax.numpy as jnp
from jax import lax
from jax.experimental import pallas as pl
from jax.experimental.pallas import tpu as pltpu
from jax.experimental.pallas import tpu_sc as plsc

N = 10000
E = 160000
D = 128
NL = 2

NP = 10240
BN = 128
BE = 640
CHUNK = 128
NCHUNK = E // CHUNK
NWORK = 32
SLOTS = (NCHUNK + NWORK - 1) // NWORK
ROWS_PER_SUB = NP // 16

_mesh = plsc.VectorSubcoreMesh(
    core_axis_name="c", subcore_axis_name="s", num_cores=2, num_subcores=16)


def _enc_node_body(feats_ref, tab_ref, out_ref):
    f = feats_ref[...]
    iota = lax.broadcasted_iota(jnp.int32, (BN, 16), 1)
    oh = jnp.concatenate(
        [(f[:, j:j + 1] == iota).astype(jnp.float32) for j in range(9)], axis=1)
    out_ref[...] = jnp.dot(oh, tab_ref[...], preferred_element_type=jnp.float32)


def _encode_nodes(feats, tab_flat):
    return pl.pallas_call(
        _enc_node_body,
        grid=(NP // BN,),
        in_specs=[pl.BlockSpec((BN, 9), lambda i: (i, 0)),
                  pl.BlockSpec((144, 128), lambda i: (0, 0))],
        out_specs=pl.BlockSpec((BN, 128), lambda i: (i, 0)),
        out_shape=jax.ShapeDtypeStruct((NP, 128), jnp.float32),
    )(feats, tab_flat)


def _enc_edge_body(feats_ref, tab_ref, We_ref, ae_ref, e_ref, hee_ref, cs_ref):
    f = feats_ref[...]
    iota = lax.broadcasted_iota(jnp.int32, (BE, 16), 1)
    oh = jnp.concatenate(
        [(f[:, j:j + 1] == iota).astype(jnp.float32) for j in range(3)], axis=1)
    e = jnp.dot(oh, tab_ref[...], preferred_element_type=jnp.float32)
    e_ref[...] = e
    va = jnp.dot(We_ref[...], ae_ref[...], preferred_element_type=jnp.float32)
    hee_ref[...] = jnp.dot(e, va, preferred_element_type=jnp.float32)
    part = jnp.sum(e, axis=0, keepdims=True)
    i = pl.program_id(0)

    @pl.when(i == 0)
    def _():
        cs_ref[...] = part

    @pl.when(i > 0)
    def _():
        cs_ref[...] = cs_ref[...] + part


def _encode_edges(feats, tab_flat, We, ae):
    return pl.pallas_call(
        _enc_edge_body,
        grid=(E // BE,),
        in_specs=[pl.BlockSpec((BE, 3), lambda i: (i, 0)),
                  pl.BlockSpec((48, 128), lambda i: (0, 0)),
                  pl.BlockSpec((128, 128), lambda i: (0, 0)),
                  pl.BlockSpec((128, 1), lambda i: (0, 0))],
        out_specs=[pl.BlockSpec((BE, 128), lambda i: (i, 0)),
                   pl.BlockSpec((BE, 1), lambda i: (i, 0)),
                   pl.BlockSpec((1, 128), lambda i: (0, 0))],
        out_shape=[jax.ShapeDtypeStruct((E, 128), jnp.float32),
                   jax.ShapeDtypeStruct((E, 1), jnp.float32),
                   jax.ShapeDtypeStruct((1, 128), jnp.float32)],
    )(feats, tab_flat, We, ae)


def _node_pre_body(x_ref, W_ref, as_ref, ad_ref, h_ref, hs_ref, hd_ref):
    h = jnp.dot(x_ref[...], W_ref[...], preferred_element_type=jnp.float32)
    h_ref[...] = h
    hs_ref[...] = jnp.dot(h, as_ref[...], preferred_element_type=jnp.float32)
    hd_ref[...] = jnp.dot(h, ad_ref[...], preferred_element_type=jnp.float32)


def _node_pre(x, W, a_s, a_d):
    return pl.pallas_call(
        _node_pre_body,
        grid=(NP // BN,),
        in_specs=[pl.BlockSpec((BN, 128), lambda i: (i, 0)),
                  pl.BlockSpec((128, 128), lambda i: (0, 0)),
                  pl.BlockSpec((128, 1), lambda i: (0, 0)),
                  pl.BlockSpec((128, 1), lambda i: (0, 0))],
        out_specs=[pl.BlockSpec((BN, 128), lambda i: (i, 0)),
                   pl.BlockSpec((BN, 1), lambda i: (i, 0)),
                   pl.BlockSpec((BN, 1), lambda i: (i, 0))],
        out_shape=[jax.ShapeDtypeStruct((NP, 128), jnp.float32),
                   jax.ShapeDtypeStruct((NP, 1), jnp.float32),
                   jax.ShapeDtypeStruct((NP, 1), jnp.float32)],
    )(x, W, a_s, a_d)


def _gat_epi_body(S_ref, den_ref, h_ref, hs_ref, hd_ref, x_ref, cs_ref,
                  We_ref, ae_ref, cb_ref, g_ref, b_ref, W1b_ref,
                  xn_ref, u_ref):
    S = S_ref[0] + S_ref[1]
    dsum = jnp.sum(den_ref[...], axis=0)[:, None]
    va = jnp.dot(We_ref[...], ae_ref[...], preferred_element_type=jnp.float32)
    c0 = jnp.dot(cs_ref[...] * (1.0 / E), va,
                 preferred_element_type=jnp.float32)
    asf = hs_ref[...] + hd_ref[...] + c0
    asf = jnp.where(asf > 0, asf, 0.2 * asf)
    w = jnp.exp(asf)
    h = h_ref[...]
    num = S + h * w
    den = dsum + w + 1e-16
    y = num / den + cb_ref[...]
    m = jnp.mean(y, axis=1, keepdims=True)
    v = jnp.mean((y - m) * (y - m), axis=1, keepdims=True)
    yn = (y - m) / jnp.sqrt(v + 1e-5) * g_ref[...] + b_ref[...]
    xn = jnp.maximum(yn, 0.0) + x_ref[...]
    xn_ref[...] = xn
    u_ref[...] = jnp.dot(xn, W1b_ref[...], preferred_element_type=jnp.float32)


def _gat_epi(Sout, den, h, hs, hd, x, cs, We, ae, cb, g, b, W1b):
    return pl.pallas_call(
        _gat_epi_body,
        grid=(NP // BN,),
        in_specs=[pl.BlockSpec((2, BN, 128), lambda i: (0, i, 0)),
                  pl.BlockSpec((NWORK, BN), lambda i: (0, i)),
                  pl.BlockSpec((BN, 128), lambda i: (i, 0)),
                  pl.BlockSpec((BN, 1), lambda i: (i, 0)),
                  pl.BlockSpec((BN, 1), lambda i: (i, 0)),
                  pl.BlockSpec((BN, 128), lambda i: (i, 0)),
                  pl.BlockSpec((1, 128), lambda i: (0, 0)),
                  pl.BlockSpec((128, 128), lambda i: (0, 0)),
                  pl.BlockSpec((128, 1), lambda i: (0, 0)),
                  pl.BlockSpec((1, 128), lambda i: (0, 0)),
                  pl.BlockSpec((1, 128), lambda i: (0, 0)),
                  pl.BlockSpec((1, 128), lambda i: (0, 0)),
                  pl.BlockSpec((128, 128), lambda i: (0, 0))],
        out_specs=[pl.BlockSpec((BN, 128), lambda i: (i, 0)),
                   pl.BlockSpec((BN, 128), lambda i: (i, 0))],
        out_shape=[jax.ShapeDtypeStruct((NP, 128), jnp.float32),
                   jax.ShapeDtypeStruct((NP, 128), jnp.float32)],
    )(Sout, den, h, hs, hd, x, cs, We, ae, cb, g, b, W1b)


def _edge_mlp_body(e_ref, z_ref, W1a_ref, b1_ref, W2_ref, b2_ref, g_ref,
                   b_ref, Wen_ref, aen_ref, en_ref, hee_ref, cs_ref):
    e = e_ref[...]
    bf = jnp.bfloat16
    h1 = jnp.maximum(
        jnp.dot(e.astype(bf), W1a_ref[...].astype(bf),
                preferred_element_type=jnp.float32)
        + z_ref[...] + b1_ref[...], 0.0)
    v = jnp.dot(h1.astype(bf), W2_ref[...].astype(bf),
                preferred_element_type=jnp.float32) + b2_ref[...]
    m = jnp.mean(v, axis=1, keepdims=True)
    var = jnp.mean((v - m) * (v - m), axis=1, keepdims=True)
    er = (v - m) / jnp.sqrt(var + 1e-5) * g_ref[...] + b_ref[...]
    en = jnp.maximum(er, 0.0) + e
    en_ref[...] = en
    va = jnp.dot(Wen_ref[...], aen_ref[...],
                 preferred_element_type=jnp.float32)
    hee_ref[...] = jnp.dot(en, va, preferred_element_type=jnp.float32)
    part = jnp.sum(en, axis=0, keepdims=True)
    i = pl.program_id(0)

    @pl.when(i == 0)
    def _():
        cs_ref[...] = part

    @pl.when(i > 0)
    def _():
        cs_ref[...] = cs_ref[...] + part


def _edge_mlp(e, z2, W1a, b1, W2, b2, g, b, Wen, aen):
    return pl.pallas_call(
        _edge_mlp_body,
        grid=(E // BE,),
        in_specs=[pl.BlockSpec((BE, 128), lambda i: (i, 0)),
                  pl.BlockSpec((BE, 128), lambda i: (i, 0)),
                  pl.BlockSpec((128, 128), lambda i: (0, 0)),
                  pl.BlockSpec((1, 128), lambda i: (0, 0)),
                  pl.BlockSpec((128, 128), lambda i: (0, 0)),
                  pl.BlockSpec((1, 128), lambda i: (0, 0)),
                  pl.BlockSpec((1, 128), lambda i: (0, 0)),
                  pl.BlockSpec((1, 128), lambda i: (0, 0)),
                  pl.BlockSpec((128, 128), lambda i: (0, 0)),
                  pl.BlockSpec((128, 1), lambda i: (0, 0))],
        out_specs=[pl.BlockSpec((BE, 128), lambda i: (i, 0)),
                   pl.BlockSpec((BE, 1), lambda i: (i, 0)),
                   pl.BlockSpec((1, 128), lambda i: (0, 0))],
        out_shape=[jax.ShapeDtypeStruct((E, 128), jnp.float32),
                   jax.ShapeDtypeStruct((E, 1), jnp.float32),
                   jax.ShapeDtypeStruct((1, 128), jnp.float32)],
    )(e, z2, W1a, b1, W2, b2, g, b, Wen, aen)


@functools.partial(
    pl.kernel,
    out_type=[jax.ShapeDtypeStruct((2, NP, 128), jnp.float32),
              jax.ShapeDtypeStruct((NWORK, NP), jnp.float32)],
    mesh=_mesh,
    scratch_types=[
        pltpu.VMEM((2, CHUNK), jnp.int32),
        pltpu.VMEM((2, CHUNK), jnp.int32),
        pltpu.VMEM((2, CHUNK), jnp.float32),
        pltpu.VMEM((2, CHUNK), jnp.float32),
        pltpu.VMEM((2, CHUNK), jnp.float32),
        pltpu.VMEM((CHUNK,), jnp.float32),
        pltpu.VMEM((2, CHUNK, 128), jnp.float32),
        pltpu.VMEM((NP,), jnp.float32),
        pltpu.VMEM_SHARED((NP, 128), jnp.float32),
        pltpu.SemaphoreType.DMA,
        pltpu.SemaphoreType.DMA,
        pltpu.SemaphoreType.DMA,
        pltpu.SemaphoreType.DMA,
        pltpu.SemaphoreType.DMA,
        pltpu.SemaphoreType.DMA,
    ],
)
def _gat_sc(src_hbm, dst_hbm, hs_hbm, hd_hbm, hee_hbm, h_hbm,
            out_hbm, den_hbm,
            is_v, id_v, he_v, hs_c, hd_c, w_v, rows_v, den_v, S_sh,
            sa1, sa2, sa3, sb1, sb2, sb3):
    c = lax.axis_index("c")
    s = lax.axis_index("s")
    wid = s * 2 + c

    zero16 = jnp.zeros((16,), jnp.float32)
    lane_iota = lax.iota(jnp.int32, 16)

    def zb(e2, carry):
        for sub in range(8):
            rows_v[0, e2, pl.ds(sub * 16, 16)] = zero16
        return carry
    lax.fori_loop(0, CHUNK, zb, 0)

    def zd(r, carry):
        den_v[pl.ds(r * 16, 16)] = zero16
        return carry
    lax.fori_loop(0, NP // 16, zd, 0)

    def zs(r, carry):
        pltpu.sync_copy(
            rows_v.at[0],
            S_sh.at[pl.ds(s * ROWS_PER_SUB + r * CHUNK, CHUNK)])
        return carry
    lax.fori_loop(0, ROWS_PER_SUB // CHUNK, zs, 0)

    plsc.subcore_barrier()

    def issue_idx(cidx, b, s1, s2, s3):
        base = cidx * CHUNK
        return (pltpu.async_copy(src_hbm.at[pl.ds(base, CHUNK)],
                                 is_v.at[b], s1),
                pltpu.async_copy(dst_hbm.at[pl.ds(base, CHUNK)],
                                 id_v.at[b], s2),
                pltpu.async_copy(hee_hbm.at[pl.ds(base, CHUNK)],
                                 he_v.at[b], s3))

    def issue_gathers(b, s1, s2, s3):
        return (pltpu.async_copy(h_hbm.at[is_v.at[b]], rows_v.at[b], s1),
                pltpu.async_copy(hs_hbm.at[is_v.at[b]], hs_c.at[b], s2),
                pltpu.async_copy(hd_hbm.at[id_v.at[b]], hd_c.at[b], s3))

    def compute(b):
        def wgrp(i, c2):
            sl = pl.ds(i * 16, 16)
            al = hs_c[b, sl] + hd_c[b, sl] + he_v[b, sl]
            al = jnp.where(al > 0, al, al * 0.2)
            w_v[sl] = jnp.exp(al)
            return c2
        lax.fori_loop(0, CHUNK // 16, wgrp, 0)

        def rowm(g, c3):
            w16 = w_v[pl.ds(g * 16, 16)]
            i16 = id_v[b, pl.ds(g * 16, 16)]
            for j in range(16):
                e2 = g * 16 + j
                wv = jnp.full((16,), w16[j], jnp.float32)
                for sub in range(8):
                    sl = pl.ds(sub * 16, 16)
                    rows_v[b, e2, sl] = rows_v[b, e2, sl] * wv
                d = i16[j]
                grp = (d >> 4) << 4
                off = jnp.full((16,), d & 15, jnp.int32)
                cur = den_v[pl.ds(grp, 16)]
                den_v[pl.ds(grp, 16)] = cur + jnp.where(
                    lane_iota == off, wv, zero16)
            return c3
        lax.fori_loop(0, CHUNK // 16, rowm, 0)
        pltpu.sync_copy(rows_v.at[b], S_sh.at[id_v.at[b]], add=True)

    npairs = (NCHUNK // NWORK) // 2

    def pair_body(p, carry):
        c0 = wid + (2 * p) * NWORK
        c1 = wid + (2 * p + 1) * NWORK
        iA = issue_idx(c0, 0, sa1, sa2, sa3)
        iB = issue_idx(c1, 1, sb1, sb2, sb3)
        for dmae in iA:
            dmae.wait()
        gA = issue_gathers(0, sa1, sa2, sa3)
        for dmae in iB:
            dmae.wait()
        gB = issue_gathers(1, sb1, sb2, sb3)
        for dmae in gA:
            dmae.wait()
        compute(0)
        for dmae in gB:
            dmae.wait()
        compute(1)
        return carry
    lax.fori_loop(0, npairs, pair_body, 0)

    def tail_body(k, carry):
        cidx = wid + k * NWORK

        @pl.when(cidx < NCHUNK)
        def _():
            iA = issue_idx(cidx, 0, sa1, sa2, sa3)
            for dmae in iA:
                dmae.wait()
            gA = issue_gathers(0, sa1, sa2, sa3)
            for dmae in gA:
                dmae.wait()
            compute(0)
        return carry
    lax.fori_loop(2 * npairs, SLOTS, tail_body, 0)

    plsc.subcore_barrier()

    def flush(r, carry):
        start = s * ROWS_PER_SUB + r * CHUNK
        pltpu.sync_copy(S_sh.at[pl.ds(start, CHUNK)],
                        out_hbm.at[c, pl.ds(start, CHUNK)])
        return carry
    lax.fori_loop(0, ROWS_PER_SUB // CHUNK, flush, 0)
    pltpu.sync_copy(den_v, den_hbm.at[wid])


@functools.partial(
    pl.kernel,
    out_type=jax.ShapeDtypeStruct((E, 128), jnp.float32),
    mesh=_mesh,
    scratch_types=[
        pltpu.VMEM((2, CHUNK), jnp.int32),
        pltpu.VMEM((2, CHUNK), jnp.int32),
        pltpu.VMEM((2, CHUNK, 128), jnp.float32),
        pltpu.SemaphoreType.DMA,
        pltpu.SemaphoreType.DMA,
        pltpu.SemaphoreType.DMA,
        pltpu.SemaphoreType.DMA,
    ],
)
def _zgather_sc(src_hbm, dst_hbm, u_hbm, out_hbm, ia_v, ib_v, ra_v,
                sa1, sa2, sb1, sb2):
    c = lax.axis_index("c")
    s = lax.axis_index("s")
    wid = s * 2 + c

    def issue_idx(cidx, b, s1, s2):
        base = cidx * CHUNK
        return (pltpu.async_copy(src_hbm.at[pl.ds(base, CHUNK)],
                                 ia_v.at[b], s1),
                pltpu.async_copy(dst_hbm.at[pl.ds(base, CHUNK)],
                                 ib_v.at[b], s2))

    def writeback(cidx, b):
        pltpu.sync_copy(ra_v.at[b], out_hbm.at[pl.ds(cidx * CHUNK, CHUNK)])

    npairs = (NCHUNK // NWORK) // 2

    def pair_body(p, carry):
        c0 = wid + (2 * p) * NWORK
        c1 = wid + (2 * p + 1) * NWORK
        iA = issue_idx(c0, 0, sa1, sa2)
        iB = issue_idx(c1, 1, sb1, sb2)
        for dmae in iA:
            dmae.wait()
        d1A = pltpu.async_copy(u_hbm.at[ia_v.at[0]], ra_v.at[0], sa1)
        for dmae in iB:
            dmae.wait()
        d1B = pltpu.async_copy(u_hbm.at[ia_v.at[1]], ra_v.at[1], sb1)
        d1A.wait()
        d2A = pltpu.async_copy(u_hbm.at[ib_v.at[0]], ra_v.at[0], sa2,
                               add=True)
        d1B.wait()
        d2B = pltpu.async_copy(u_hbm.at[ib_v.at[1]], ra_v.at[1], sb2,
                               add=True)
        d2A.wait()
        writeback(c0, 0)
        d2B.wait()
        writeback(c1, 1)
        return carry
    lax.fori_loop(0, npairs, pair_body, 0)

    def tail_body(k, carry):
        cidx = wid + k * NWORK

        @pl.when(cidx < NCHUNK)
        def _():
            iA = issue_idx(cidx, 0, sa1, sa2)
            for dmae in iA:
                dmae.wait()
            d1 = pltpu.async_copy(u_hbm.at[ia_v.at[0]], ra_v.at[0], sa1)
            d1.wait()
            d2 = pltpu.async_copy(u_hbm.at[ib_v.at[0]], ra_v.at[0], sa2,
                                  add=True)
            d2.wait()
            writeback(cidx, 0)
        return carry
    lax.fori_loop(2 * npairs, SLOTS, tail_body, 0)


def kernel(reac_x, reac_edge_index, reac_edge_attr, prod_x, prod_edge_index,
           prod_edge_attr, atom_tables, bond_tables, conv_W, conv_as, conv_ad,
           conv_We, conv_ae, conv_b, bn_g, bn_b, eu_W1, eu_b1, eu_W2, eu_b2,
           ln_g, ln_b):
    f32 = jnp.float32
    atom_flat = atom_tables.reshape(9 * 16, 128).astype(f32)
    bond_flat = bond_tables.reshape(3 * 16, 128).astype(f32)

    def pad_feats(fx):
        return jnp.pad(fx.astype(jnp.int32), ((0, NP - N), (0, 0)))

    graphs = {}
    for g, (fx, ei, ea) in (("r", (reac_x, reac_edge_index, reac_edge_attr)),
                            ("p", (prod_x, prod_edge_index, prod_edge_attr))):
        gi = 0 if g == "r" else 1
        l0 = gi
        src = ei[0].astype(jnp.int32)
        dst = ei[1].astype(jnp.int32)
        x = _encode_nodes(pad_feats(fx), atom_flat)
        e, hee, cs = _encode_edges(
            ea.astype(jnp.int32), bond_flat, conv_We[l0],
            conv_ae[l0].reshape(128, 1))
        graphs[g] = dict(x=x, e=e, src=src, dst=dst,
                         hee=hee.reshape(-1), cs=cs)

    for i in range(NL):
        for gi, g in enumerate(("r", "p")):
            l = 2 * i + gi
            ln_next = 2 * (i + 1) + gi if i + 1 < NL else l
            st = graphs[g]
            h, hs, hd = _node_pre(st["x"], conv_W[l],
                                  conv_as[l].reshape(128, 1),
                                  conv_ad[l].reshape(128, 1))
            Sout, den = _gat_sc(st["src"], st["dst"], hs.reshape(-1),
                                hd.reshape(-1), st["hee"], h)
            xn, u = _gat_epi(Sout, den, h, hs, hd, st["x"], st["cs"],
                             conv_We[l],
                             conv_ae[l].reshape(128, 1),
                             conv_b[l].reshape(1, 128),
                             bn_g[l].reshape(1, 128),
                             bn_b[l].reshape(1, 128), eu_W1[l][D:])
            z2 = _zgather_sc(st["src"], st["dst"], u)
            en, hee_n, cs_n = _edge_mlp(
                st["e"], z2, eu_W1[l][:D], eu_b1[l].reshape(1, 128),
                eu_W2[l], eu_b2[l].reshape(1, 128), ln_g[l].reshape(1, 128),
                ln_b[l].reshape(1, 128), conv_We[ln_next],
                conv_ae[ln_next].reshape(128, 1))
            st.update(x=xn, e=en, hee=hee_n.reshape(-1), cs=cs_n)

    return (graphs["r"]["x"][:N], graphs["p"]["x"][:N],
            graphs["r"]["e"], graphs["p"]["e"])


# speedup vs baseline: 1.0024x; 1.0024x over previous
import functools

import jax
import jax.numpy as jnp
from jax import lax
from jax.experimental import pallas as pl
from jax.experimental.pallas import tpu as pltpu
from jax.experimental.pallas import tpu_sc as plsc

N = 10000
E = 160000
D = 128
NL = 2

NP = 10240
BN = 128
BE = 640
CHUNK = 128
NCHUNK = E // CHUNK
NWORK = 32
SLOTS = (NCHUNK + NWORK - 1) // NWORK
ROWS_PER_SUB = NP // 16

_mesh = plsc.VectorSubcoreMesh(
    core_axis_name="c", subcore_axis_name="s", num_cores=2, num_subcores=16)


def _enc_node_body(feats_ref, tab_ref, out_ref):
    f = feats_ref[...]
    iota = lax.broadcasted_iota(jnp.int32, (BN, 16), 1)
    oh = jnp.concatenate(
        [(f[:, j:j + 1] == iota).astype(jnp.float32) for j in range(9)], axis=1)
    out_ref[...] = jnp.dot(oh, tab_ref[...], preferred_element_type=jnp.float32)


def _encode_nodes(feats, tab_flat):
    return pl.pallas_call(
        _enc_node_body,
        grid=(NP // BN,),
        in_specs=[pl.BlockSpec((BN, 9), lambda i: (i, 0)),
                  pl.BlockSpec((144, 128), lambda i: (0, 0))],
        out_specs=pl.BlockSpec((BN, 128), lambda i: (i, 0)),
        out_shape=jax.ShapeDtypeStruct((NP, 128), jnp.float32),
    )(feats, tab_flat)


def _enc_edge_body(feats_ref, tab_ref, We_ref, ae_ref, e_ref, hee_ref, cs_ref):
    f = feats_ref[...]
    iota = lax.broadcasted_iota(jnp.int32, (BE, 16), 1)
    oh = jnp.concatenate(
        [(f[:, j:j + 1] == iota).astype(jnp.float32) for j in range(3)], axis=1)
    e = jnp.dot(oh, tab_ref[...], preferred_element_type=jnp.float32)
    e_ref[...] = e
    va = jnp.dot(We_ref[...], ae_ref[...], preferred_element_type=jnp.float32)
    hee_ref[...] = jnp.dot(e, va, preferred_element_type=jnp.float32)
    part = jnp.sum(e, axis=0, keepdims=True)
    i = pl.program_id(0)

    @pl.when(i == 0)
    def _():
        cs_ref[...] = part

    @pl.when(i > 0)
    def _():
        cs_ref[...] = cs_ref[...] + part


def _encode_edges(feats, tab_flat, We, ae):
    return pl.pallas_call(
        _enc_edge_body,
        grid=(E // BE,),
        in_specs=[pl.BlockSpec((BE, 3), lambda i: (i, 0)),
                  pl.BlockSpec((48, 128), lambda i: (0, 0)),
                  pl.BlockSpec((128, 128), lambda i: (0, 0)),
                  pl.BlockSpec((128, 1), lambda i: (0, 0))],
        out_specs=[pl.BlockSpec((BE, 128), lambda i: (i, 0)),
                   pl.BlockSpec((BE, 1), lambda i: (i, 0)),
                   pl.BlockSpec((1, 128), lambda i: (0, 0))],
        out_shape=[jax.ShapeDtypeStruct((E, 128), jnp.float32),
                   jax.ShapeDtypeStruct((E, 1), jnp.float32),
                   jax.ShapeDtypeStruct((1, 128), jnp.float32)],
    )(feats, tab_flat, We, ae)


def _node_pre_body(x_ref, W_ref, as_ref, ad_ref, h_ref, hs_ref, hd_ref):
    h = jnp.dot(x_ref[...], W_ref[...], preferred_element_type=jnp.float32)
    h_ref[...] = h
    hs_ref[...] = jnp.dot(h, as_ref[...], preferred_element_type=jnp.float32)
    hd_ref[...] = jnp.dot(h, ad_ref[...], preferred_element_type=jnp.float32)


def _node_pre(x, W, a_s, a_d):
    return pl.pallas_call(
        _node_pre_body,
        grid=(NP // BN,),
        in_specs=[pl.BlockSpec((BN, 128), lambda i: (i, 0)),
                  pl.BlockSpec((128, 128), lambda i: (0, 0)),
                  pl.BlockSpec((128, 1), lambda i: (0, 0)),
                  pl.BlockSpec((128, 1), lambda i: (0, 0))],
        out_specs=[pl.BlockSpec((BN, 128), lambda i: (i, 0)),
                   pl.BlockSpec((BN, 1), lambda i: (i, 0)),
                   pl.BlockSpec((BN, 1), lambda i: (i, 0))],
        out_shape=[jax.ShapeDtypeStruct((NP, 128), jnp.float32),
                   jax.ShapeDtypeStruct((NP, 1), jnp.float32),
                   jax.ShapeDtypeStruct((NP, 1), jnp.float32)],
    )(x, W, a_s, a_d)


def _gat_epi_body(S_ref, den_ref, h_ref, hs_ref, hd_ref, x_ref, cs_ref,
                  We_ref, ae_ref, cb_ref, g_ref, b_ref, W1b_ref,
                  xn_ref, u_ref):
    S = S_ref[0] + S_ref[1]
    dsum = jnp.sum(den_ref[...], axis=0)[:, None]
    va = jnp.dot(We_ref[...], ae_ref[...], preferred_element_type=jnp.float32)
    c0 = jnp.dot(cs_ref[...] * (1.0 / E), va,
                 preferred_element_type=jnp.float32)
    asf = hs_ref[...] + hd_ref[...] + c0
    asf = jnp.where(asf > 0, asf, 0.2 * asf)
    w = jnp.exp(asf)
    h = h_ref[...]
    num = S + h * w
    den = dsum + w + 1e-16
    y = num / den + cb_ref[...]
    m = jnp.mean(y, axis=1, keepdims=True)
    v = jnp.mean((y - m) * (y - m), axis=1, keepdims=True)
    yn = (y - m) / jnp.sqrt(v + 1e-5) * g_ref[...] + b_ref[...]
    xn = jnp.maximum(yn, 0.0) + x_ref[...]
    xn_ref[...] = xn
    u_ref[...] = jnp.dot(xn, W1b_ref[...], preferred_element_type=jnp.float32)


def _gat_epi(Sout, den, h, hs, hd, x, cs, We, ae, cb, g, b, W1b):
    return pl.pallas_call(
        _gat_epi_body,
        grid=(NP // BN,),
        in_specs=[pl.BlockSpec((2, BN, 128), lambda i: (0, i, 0)),
                  pl.BlockSpec((NWORK, BN), lambda i: (0, i)),
                  pl.BlockSpec((BN, 128), lambda i: (i, 0)),
                  pl.BlockSpec((BN, 1), lambda i: (i, 0)),
                  pl.BlockSpec((BN, 1), lambda i: (i, 0)),
                  pl.BlockSpec((BN, 128), lambda i: (i, 0)),
                  pl.BlockSpec((1, 128), lambda i: (0, 0)),
                  pl.BlockSpec((128, 128), lambda i: (0, 0)),
                  pl.BlockSpec((128, 1), lambda i: (0, 0)),
                  pl.BlockSpec((1, 128), lambda i: (0, 0)),
                  pl.BlockSpec((1, 128), lambda i: (0, 0)),
                  pl.BlockSpec((1, 128), lambda i: (0, 0)),
                  pl.BlockSpec((128, 128), lambda i: (0, 0))],
        out_specs=[pl.BlockSpec((BN, 128), lambda i: (i, 0)),
                   pl.BlockSpec((BN, 128), lambda i: (i, 0))],
        out_shape=[jax.ShapeDtypeStruct((NP, 128), jnp.float32),
                   jax.ShapeDtypeStruct((NP, 128), jnp.float32)],
    )(Sout, den, h, hs, hd, x, cs, We, ae, cb, g, b, W1b)


def _edge_mlp_body(e_ref, z_ref, W1a_ref, b1_ref, W2_ref, b2_ref, g_ref,
                   b_ref, Wen_ref, aen_ref, en_ref, hee_ref, cs_ref):
    e = e_ref[...]
    h1 = jnp.maximum(
        jnp.dot(e, W1a_ref[...], preferred_element_type=jnp.float32)
        + z_ref[...] + b1_ref[...], 0.0)
    v = jnp.dot(h1, W2_ref[...],
                preferred_element_type=jnp.float32) + b2_ref[...]
    m = jnp.mean(v, axis=1, keepdims=True)
    var = jnp.mean((v - m) * (v - m), axis=1, keepdims=True)
    er = (v - m) / jnp.sqrt(var + 1e-5) * g_ref[...] + b_ref[...]
    en = jnp.maximum(er, 0.0) + e
    en_ref[...] = en
    va = jnp.dot(Wen_ref[...], aen_ref[...],
                 preferred_element_type=jnp.float32)
    hee_ref[...] = jnp.dot(en, va, preferred_element_type=jnp.float32)
    part = jnp.sum(en, axis=0, keepdims=True)
    i = pl.program_id(0)

    @pl.when(i == 0)
    def _():
        cs_ref[...] = part

    @pl.when(i > 0)
    def _():
        cs_ref[...] = cs_ref[...] + part


def _edge_mlp(e, z2, W1a, b1, W2, b2, g, b, Wen, aen):
    return pl.pallas_call(
        _edge_mlp_body,
        grid=(E // BE,),
        in_specs=[pl.BlockSpec((BE, 128), lambda i: (i, 0)),
                  pl.BlockSpec((BE, 128), lambda i: (i, 0)),
                  pl.BlockSpec((128, 128), lambda i: (0, 0)),
                  pl.BlockSpec((1, 128), lambda i: (0, 0)),
                  pl.BlockSpec((128, 128), lambda i: (0, 0)),
                  pl.BlockSpec((1, 128), lambda i: (0, 0)),
                  pl.BlockSpec((1, 128), lambda i: (0, 0)),
                  pl.BlockSpec((1, 128), lambda i: (0, 0)),
                  pl.BlockSpec((128, 128), lambda i: (0, 0)),
                  pl.BlockSpec((128, 1), lambda i: (0, 0))],
        out_specs=[pl.BlockSpec((BE, 128), lambda i: (i, 0)),
                   pl.BlockSpec((BE, 1), lambda i: (i, 0)),
                   pl.BlockSpec((1, 128), lambda i: (0, 0))],
        out_shape=[jax.ShapeDtypeStruct((E, 128), jnp.float32),
                   jax.ShapeDtypeStruct((E, 1), jnp.float32),
                   jax.ShapeDtypeStruct((1, 128), jnp.float32)],
    )(e, z2, W1a, b1, W2, b2, g, b, Wen, aen)


@functools.partial(
    pl.kernel,
    out_type=[jax.ShapeDtypeStruct((2, NP, 128), jnp.float32),
              jax.ShapeDtypeStruct((NWORK, NP), jnp.float32)],
    mesh=_mesh,
    scratch_types=[
        pltpu.VMEM((2, CHUNK), jnp.int32),
        pltpu.VMEM((2, CHUNK), jnp.int32),
        pltpu.VMEM((2, CHUNK), jnp.float32),
        pltpu.VMEM((2, CHUNK), jnp.float32),
        pltpu.VMEM((2, CHUNK), jnp.float32),
        pltpu.VMEM((CHUNK,), jnp.float32),
        pltpu.VMEM((2, CHUNK, 128), jnp.float32),
        pltpu.VMEM((NP,), jnp.float32),
        pltpu.VMEM_SHARED((NP, 128), jnp.float32),
        pltpu.SemaphoreType.DMA,
        pltpu.SemaphoreType.DMA,
        pltpu.SemaphoreType.DMA,
        pltpu.SemaphoreType.DMA,
        pltpu.SemaphoreType.DMA,
        pltpu.SemaphoreType.DMA,
    ],
)
def _gat_sc(src_hbm, dst_hbm, hs_hbm, hd_hbm, hee_hbm, h_hbm,
            out_hbm, den_hbm,
            is_v, id_v, he_v, hs_c, hd_c, w_v, rows_v, den_v, S_sh,
            sa1, sa2, sa3, sb1, sb2, sb3):
    c = lax.axis_index("c")
    s = lax.axis_index("s")
    wid = s * 2 + c

    zero16 = jnp.zeros((16,), jnp.float32)
    lane_iota = lax.iota(jnp.int32, 16)

    def zb(e2, carry):
        for sub in range(8):
            rows_v[0, e2, pl.ds(sub * 16, 16)] = zero16
        return carry
    lax.fori_loop(0, CHUNK, zb, 0)

    def zd(r, carry):
        den_v[pl.ds(r * 16, 16)] = zero16
        return carry
    lax.fori_loop(0, NP // 16, zd, 0)

    def zs(r, carry):
        pltpu.sync_copy(
            rows_v.at[0],
            S_sh.at[pl.ds(s * ROWS_PER_SUB + r * CHUNK, CHUNK)])
        return carry
    lax.fori_loop(0, ROWS_PER_SUB // CHUNK, zs, 0)

    plsc.subcore_barrier()

    def issue_idx(cidx, b, s1, s2, s3):
        base = cidx * CHUNK
        return (pltpu.async_copy(src_hbm.at[pl.ds(base, CHUNK)],
                                 is_v.at[b], s1),
                pltpu.async_copy(dst_hbm.at[pl.ds(base, CHUNK)],
                                 id_v.at[b], s2),
                pltpu.async_copy(hee_hbm.at[pl.ds(base, CHUNK)],
                                 he_v.at[b], s3))

    def issue_gathers(b, s1, s2, s3):
        return (pltpu.async_copy(h_hbm.at[is_v.at[b]], rows_v.at[b], s1),
                pltpu.async_copy(hs_hbm.at[is_v.at[b]], hs_c.at[b], s2),
                pltpu.async_copy(hd_hbm.at[id_v.at[b]], hd_c.at[b], s3))

    def compute(b):
        def wgrp(i, c2):
            sl = pl.ds(i * 16, 16)
            al = hs_c[b, sl] + hd_c[b, sl] + he_v[b, sl]
            al = jnp.where(al > 0, al, al * 0.2)
            w_v[sl] = jnp.exp(al)
            return c2
        lax.fori_loop(0, CHUNK // 16, wgrp, 0)

        def rowm(g, c3):
            w16 = w_v[pl.ds(g * 16, 16)]
            i16 = id_v[b, pl.ds(g * 16, 16)]
            for j in range(16):
                e2 = g * 16 + j
                wv = jnp.full((16,), w16[j], jnp.float32)
                for sub in range(8):
                    sl = pl.ds(sub * 16, 16)
                    rows_v[b, e2, sl] = rows_v[b, e2, sl] * wv
                d = i16[j]
                grp = (d >> 4) << 4
                off = jnp.full((16,), d & 15, jnp.int32)
                cur = den_v[pl.ds(grp, 16)]
                den_v[pl.ds(grp, 16)] = cur + jnp.where(
                    lane_iota == off, wv, zero16)
            return c3
        lax.fori_loop(0, CHUNK // 16, rowm, 0)
        pltpu.sync_copy(rows_v.at[b], S_sh.at[id_v.at[b]], add=True)

    npairs = (NCHUNK // NWORK) // 2

    def pair_body(p, carry):
        c0 = wid + (2 * p) * NWORK
        c1 = wid + (2 * p + 1) * NWORK
        iA = issue_idx(c0, 0, sa1, sa2, sa3)
        iB = issue_idx(c1, 1, sb1, sb2, sb3)
        for dmae in iA:
            dmae.wait()
        gA = issue_gathers(0, sa1, sa2, sa3)
        for dmae in iB:
            dmae.wait()
        gB = issue_gathers(1, sb1, sb2, sb3)
        for dmae in gA:
            dmae.wait()
        compute(0)
        for dmae in gB:
            dmae.wait()
        compute(1)
        return carry
    lax.fori_loop(0, npairs, pair_body, 0)

    def tail_body(k, carry):
        cidx = wid + k * NWORK

        @pl.when(cidx < NCHUNK)
        def _():
            iA = issue_idx(cidx, 0, sa1, sa2, sa3)
            for dmae in iA:
                dmae.wait()
            gA = issue_gathers(0, sa1, sa2, sa3)
            for dmae in gA:
                dmae.wait()
            compute(0)
        return carry
    lax.fori_loop(2 * npairs, SLOTS, tail_body, 0)

    plsc.subcore_barrier()

    def flush(r, carry):
        start = s * ROWS_PER_SUB + r * CHUNK
        pltpu.sync_copy(S_sh.at[pl.ds(start, CHUNK)],
                        out_hbm.at[c, pl.ds(start, CHUNK)])
        return carry
    lax.fori_loop(0, ROWS_PER_SUB // CHUNK, flush, 0)
    pltpu.sync_copy(den_v, den_hbm.at[wid])


@functools.partial(
    pl.kernel,
    out_type=jax.ShapeDtypeStruct((E, 128), jnp.float32),
    mesh=_mesh,
    scratch_types=[
        pltpu.VMEM((2, CHUNK), jnp.int32),
        pltpu.VMEM((2, CHUNK), jnp.int32),
        pltpu.VMEM((2, CHUNK, 128), jnp.float32),
        pltpu.SemaphoreType.DMA,
        pltpu.SemaphoreType.DMA,
        pltpu.SemaphoreType.DMA,
        pltpu.SemaphoreType.DMA,
    ],
)
def _zgather_sc(src_hbm, dst_hbm, u_hbm, out_hbm, ia_v, ib_v, ra_v,
                sa1, sa2, sb1, sb2):
    c = lax.axis_index("c")
    s = lax.axis_index("s")
    wid = s * 2 + c

    def issue_idx(cidx, b, s1, s2):
        base = cidx * CHUNK
        return (pltpu.async_copy(src_hbm.at[pl.ds(base, CHUNK)],
                                 ia_v.at[b], s1),
                pltpu.async_copy(dst_hbm.at[pl.ds(base, CHUNK)],
                                 ib_v.at[b], s2))

    def writeback(cidx, b):
        pltpu.sync_copy(ra_v.at[b], out_hbm.at[pl.ds(cidx * CHUNK, CHUNK)])

    npairs = (NCHUNK // NWORK) // 2

    def pair_body(p, carry):
        c0 = wid + (2 * p) * NWORK
        c1 = wid + (2 * p + 1) * NWORK
        iA = issue_idx(c0, 0, sa1, sa2)
        iB = issue_idx(c1, 1, sb1, sb2)
        for dmae in iA:
            dmae.wait()
        d1A = pltpu.async_copy(u_hbm.at[ia_v.at[0]], ra_v.at[0], sa1)
        for dmae in iB:
            dmae.wait()
        d1B = pltpu.async_copy(u_hbm.at[ia_v.at[1]], ra_v.at[1], sb1)
        d1A.wait()
        d2A = pltpu.async_copy(u_hbm.at[ib_v.at[0]], ra_v.at[0], sa2,
                               add=True)
        d1B.wait()
        d2B = pltpu.async_copy(u_hbm.at[ib_v.at[1]], ra_v.at[1], sb2,
                               add=True)
        d2A.wait()
        writeback(c0, 0)
        d2B.wait()
        writeback(c1, 1)
        return carry
    lax.fori_loop(0, npairs, pair_body, 0)

    def tail_body(k, carry):
        cidx = wid + k * NWORK

        @pl.when(cidx < NCHUNK)
        def _():
            iA = issue_idx(cidx, 0, sa1, sa2)
            for dmae in iA:
                dmae.wait()
            d1 = pltpu.async_copy(u_hbm.at[ia_v.at[0]], ra_v.at[0], sa1)
            d1.wait()
            d2 = pltpu.async_copy(u_hbm.at[ib_v.at[0]], ra_v.at[0], sa2,
                                  add=True)
            d2.wait()
            writeback(cidx, 0)
        return carry
    lax.fori_loop(2 * npairs, SLOTS, tail_body, 0)


def kernel(reac_x, reac_edge_index, reac_edge_attr, prod_x, prod_edge_index,
           prod_edge_attr, atom_tables, bond_tables, conv_W, conv_as, conv_ad,
           conv_We, conv_ae, conv_b, bn_g, bn_b, eu_W1, eu_b1, eu_W2, eu_b2,
           ln_g, ln_b):
    f32 = jnp.float32
    atom_flat = atom_tables.reshape(9 * 16, 128).astype(f32)
    bond_flat = bond_tables.reshape(3 * 16, 128).astype(f32)

    def pad_feats(fx):
        return jnp.pad(fx.astype(jnp.int32), ((0, NP - N), (0, 0)))

    graphs = {}
    for g, (fx, ei, ea) in (("r", (reac_x, reac_edge_index, reac_edge_attr)),
                            ("p", (prod_x, prod_edge_index, prod_edge_attr))):
        gi = 0 if g == "r" else 1
        l0 = gi
        src = ei[0].astype(jnp.int32)
        dst = ei[1].astype(jnp.int32)
        x = _encode_nodes(pad_feats(fx), atom_flat)
        e, hee, cs = _encode_edges(
            ea.astype(jnp.int32), bond_flat, conv_We[l0],
            conv_ae[l0].reshape(128, 1))
        graphs[g] = dict(x=x, e=e, src=src, dst=dst,
                         hee=hee.reshape(-1), cs=cs)

    for i in range(NL):
        for gi, g in enumerate(("r", "p")):
            l = 2 * i + gi
            ln_next = 2 * (i + 1) + gi if i + 1 < NL else l
            st = graphs[g]
            h, hs, hd = _node_pre(st["x"], conv_W[l],
                                  conv_as[l].reshape(128, 1),
                                  conv_ad[l].reshape(128, 1))
            Sout, den = _gat_sc(st["src"], st["dst"], hs.reshape(-1),
                                hd.reshape(-1), st["hee"], h)
            xn, u = _gat_epi(Sout, den, h, hs, hd, st["x"], st["cs"],
                             conv_We[l],
                             conv_ae[l].reshape(128, 1),
                             conv_b[l].reshape(1, 128),
                             bn_g[l].reshape(1, 128),
                             bn_b[l].reshape(1, 128), eu_W1[l][D:])
            z2 = _zgather_sc(st["src"], st["dst"], u)
            en, hee_n, cs_n = _edge_mlp(
                st["e"], z2, eu_W1[l][:D], eu_b1[l].reshape(1, 128),
                eu_W2[l], eu_b2[l].reshape(1, 128), ln_g[l].reshape(1, 128),
                ln_b[l].reshape(1, 128), conv_We[ln_next],
                conv_ae[ln_next].reshape(128, 1))
            st.update(x=xn, e=en, hee=hee_n.reshape(-1), cs=cs_n)

    return (graphs["r"]["x"][:N], graphs["p"]["x"][:N],
            graphs["r"]["e"], graphs["p"]["e"])
